# baseline (speedup 1.0000x reference)
import jax
import jax.numpy as jnp
from jax import lax
from jax.experimental import pallas as pl
from jax.experimental.pallas import tpu as pltpu, tpu_sc as plsc

B, H, W = 4, 512, 512
HW = H * W
P = B * HW
NC, NS, L = 2, 16, 16
PIX_PER_TILE = HW // NS
CHUNK = 4096
NCHUNK = PIX_PER_TILE // CHUNK
NPAIR = 4 * CHUNK
WIN = 32
HALF = WIN // 2
TBL = WIN * WIN
SINK0 = HW
SINKW = 2 * CHUNK
ACC_WORDS = HW + NS * SINKW


def _splat_body(phi_hbm, out_hbm,
                gy0, gy1, gx0, gx1, idx0, idx1, w0, w1, tbl, midx,
                acc, sem_in0, sem_in1, sem_sc0, sem_sc1, sem_z):
    c = lax.axis_index("c")
    s = lax.axis_index("s")
    q = s * PIX_PER_TILE
    sink = SINK0 + s * SINKW
    lane = lax.iota(jnp.int32, L)
    zeros16 = jnp.zeros((L,), jnp.float32)
    laneoff = lane << 10

    gy_bufs = (gy0, gy1)
    gx_bufs = (gx0, gx1)
    idx_bufs = (idx0, idx1)
    w_bufs = (w0, w1)
    sem_ins = (sem_in0, sem_in1)
    sem_scs = (sem_sc0, sem_sc1)

    @pl.loop(0, TBL // L)
    def _init_tbl(j):
        wcell = j * L + lane
        r = wcell >> 5
        col = wcell & (WIN - 1)
        gy_ = (r - HALF) & (H - 1)
        gx_ = (col - HALF) & (W - 1)
        midx[pl.ds(j * L, L)] = (gy_ << 9) + gx_

    ztile = pl.multiple_of(s * PIX_PER_TILE, PIX_PER_TILE)

    for phase in range(2):
        b = 2 * c + phase
        gy_off = pl.multiple_of(b * (2 * HW) + q, CHUNK)
        gx_off = pl.multiple_of(b * (2 * HW) + HW + q, CHUNK)

        @pl.loop(0, (TBL * L + L) // L)
        def _ztbl(j):
            tbl[pl.ds(j * L, L)] = zeros16

        @pl.loop(0, NPAIR // L)
        def _zero(i):
            w0[pl.ds(i * L, L)] = zeros16

        z0 = pltpu.async_copy(w0, acc.at[pl.ds(ztile, PIX_PER_TILE)], sem_z)
        z0.wait()
        plsc.subcore_barrier()

        def start_inputs(ch):
            d = ch & 1
            a = pltpu.async_copy(
                phi_hbm.at[pl.ds(gy_off + ch * CHUNK, CHUNK)],
                gy_bufs[d], sem_ins[d])
            bcp = pltpu.async_copy(
                phi_hbm.at[pl.ds(gx_off + ch * CHUNK, CHUNK)],
                gx_bufs[d], sem_ins[d])
            return (a, bcp)

        in_pend = {0: start_inputs(0), 1: start_inputs(1)}
        sc_pend = {}

        for ch in range(NCHUNK):
            d = ch & 1
            for cp in in_pend.pop(ch):
                cp.wait()
            if ch - 2 in sc_pend:
                sc_pend.pop(ch - 2).wait()
            gy_buf, gx_buf = gy_bufs[d], gx_bufs[d]
            idx_buf, w_buf = idx_bufs[d], w_bufs[d]

            @pl.loop(0, CHUNK // L)
            def _compute(i):
                gy = gy_buf[pl.ds(i * L, L)]
                gx = gx_buf[pl.ds(i * L, L)]
                ty = gy.astype(jnp.int32)
                tx = gx.astype(jnp.int32)
                tyf = ty.astype(jnp.float32)
                txf = tx.astype(jnp.float32)
                cy = tyf > gy
                cx = txf > gx
                y0i = jnp.where(cy, ty - 1, ty)
                x0i = jnp.where(cx, tx - 1, tx)
                wy = gy - jnp.where(cy, tyf - 1.0, tyf)
                wx = gx - jnp.where(cx, txf - 1.0, txf)
                uy = 1.0 - wy
                ux = 1.0 - wx
                y0 = y0i & (H - 1)
                x0 = x0i & (W - 1)
                y1 = (y0i + 1) & (H - 1)
                x1 = (x0i + 1) & (W - 1)
                w00 = uy * ux
                w01 = uy * wx
                w10 = wy * ux
                w11 = wy * wx
                yy0 = (y0 + HALF) & (H - 1)
                yy1 = (y1 + HALF) & (H - 1)
                xx0 = (x0 + HALF) & (W - 1)
                xx1 = (x1 + HALF) & (W - 1)
                iny0 = yy0 < WIN
                iny1 = yy1 < WIN
                inx0 = xx0 < WIN
                inx1 = xx1 < WIN
                m00 = iny0 & inx0
                m01 = iny0 & inx1
                m10 = iny1 & inx0
                m11 = iny1 & inx1
                ry0 = yy0 << 5
                ry1 = yy1 << 5
                dump = TBL * L + lane

                def local_add(m, cell, w):
                    l = jnp.where(m, laneoff + cell, dump)
                    cur = plsc.load_gather(tbl, [l])
                    plsc.store_scatter(tbl, [l], cur + jnp.where(m, w, 0.0))

                local_add(m00, ry0 + xx0, w00)
                local_add(m01, ry0 + xx1, w01)
                local_add(m10, ry1 + xx0, w10)
                local_add(m11, ry1 + xx1, w11)
                r0 = y0 << 9
                r1 = y1 << 9
                o = i * L
                p0 = sink + o + lane
                fz = jnp.float32(0.0)
                idx_buf[pl.ds(o, L)] = jnp.where(m00, p0, r0 + x0)
                idx_buf[pl.ds(CHUNK + o, L)] = jnp.where(m01, p0 + CHUNK, r0 + x1)
                idx_buf[pl.ds(2 * CHUNK + o, L)] = jnp.where(m10, p0, r1 + x0)
                idx_buf[pl.ds(3 * CHUNK + o, L)] = jnp.where(m11, p0 + CHUNK, r1 + x1)
                w_buf[pl.ds(o, L)] = jnp.where(m00, fz, w00)
                w_buf[pl.ds(CHUNK + o, L)] = jnp.where(m01, fz, w01)
                w_buf[pl.ds(2 * CHUNK + o, L)] = jnp.where(m10, fz, w10)
                w_buf[pl.ds(3 * CHUNK + o, L)] = jnp.where(m11, fz, w11)

            sc_pend[ch] = pltpu.async_copy(
                w_buf, acc.at[idx_buf], sem_scs[d], add=True)
            if ch + 2 < NCHUNK:
                in_pend[ch + 2] = start_inputs(ch + 2)

        for ch in sorted(sc_pend):
            sc_pend.pop(ch).wait()

        @pl.loop(0, TBL // L)
        def _merge(j):
            v = tbl[pl.ds(TBL + j * L, L)]
            for k in range(2, L):
                v = v + tbl[pl.ds(k * TBL + j * L, L)]
            tbl[pl.ds(j * L, L)] = tbl[pl.ds(j * L, L)] + v

        pltpu.sync_copy(tbl.at[pl.ds(0, TBL)], acc.at[midx], add=True)
        plsc.subcore_barrier()

        pltpu.sync_copy(
            acc.at[pl.ds(ztile, PIX_PER_TILE)],
            out_hbm.at[pl.ds(pl.multiple_of(b * HW + q, PIX_PER_TILE),
                             PIX_PER_TILE)],
        )


def _make_splat():
    mesh = plsc.VectorSubcoreMesh(core_axis_name="c", subcore_axis_name="s")
    return pl.kernel(
        _splat_body,
        out_type=jax.ShapeDtypeStruct((P,), jnp.float32),
        mesh=mesh,
        compiler_params=pltpu.CompilerParams(needs_layout_passes=False),
        scratch_types=[
            pltpu.VMEM((CHUNK,), jnp.float32),
            pltpu.VMEM((CHUNK,), jnp.float32),
            pltpu.VMEM((CHUNK,), jnp.float32),
            pltpu.VMEM((CHUNK,), jnp.float32),
            pltpu.VMEM((NPAIR,), jnp.int32),
            pltpu.VMEM((NPAIR,), jnp.int32),
            pltpu.VMEM((NPAIR,), jnp.float32),
            pltpu.VMEM((NPAIR,), jnp.float32),
            pltpu.VMEM((TBL * L + L,), jnp.float32),
            pltpu.VMEM((TBL,), jnp.int32),
            pltpu.VMEM_SHARED((ACC_WORDS,), jnp.float32),
            pltpu.SemaphoreType.DMA,
            pltpu.SemaphoreType.DMA,
            pltpu.SemaphoreType.DMA,
            pltpu.SemaphoreType.DMA,
            pltpu.SemaphoreType.DMA,
        ],
    )


_splat = _make_splat()


@jax.jit
def kernel(x, phi):
    del x
    cnt = _splat(phi.reshape(-1))
    return cnt.reshape(B, 1, H, W)


# speedup vs baseline: 1.2761x; 1.2761x over previous
import jax
import jax.numpy as jnp
from jax import lax
from jax.experimental import pallas as pl
from jax.experimental.pallas import tpu as pltpu, tpu_sc as plsc

B, H, W = 4, 512, 512
HW = H * W
P = B * HW
NC, NS, L = 2, 16, 16
PIX_PER_TILE = HW // NS
CHUNK = 4096
NCHUNK = PIX_PER_TILE // CHUNK
NPAIR = 4 * CHUNK
WIN = 16
HALF = WIN // 2
TBL = WIN * WIN
NTBL = 4
TWORDS = TBL * L + L
SINK0 = HW
SINKW = 2 * CHUNK
ACC_WORDS = HW + NS * SINKW


def _splat_body(phi_hbm, out_hbm,
                gy0, gy1, gx0, gx1, idx0, idx1, w0, w1,
                tb0, tb1, tb2, tb3, midx,
                acc, sem_in0, sem_in1, sem_sc0, sem_sc1, sem_z):
    c = lax.axis_index("c")
    s = lax.axis_index("s")
    q = s * PIX_PER_TILE
    sink = SINK0 + s * SINKW
    lane = lax.iota(jnp.int32, L)
    zeros16 = jnp.zeros((L,), jnp.float32)
    laneoff = lane << 8

    gy_bufs = (gy0, gy1)
    gx_bufs = (gx0, gx1)
    idx_bufs = (idx0, idx1)
    w_bufs = (w0, w1)
    sem_ins = (sem_in0, sem_in1)
    sem_scs = (sem_sc0, sem_sc1)
    tbs = (tb0, tb1, tb2, tb3)

    @pl.loop(0, TBL // L)
    def _init_tbl(j):
        wcell = j * L + lane
        r = wcell >> 4
        col = wcell & (WIN - 1)
        gy_ = (r - HALF) & (H - 1)
        gx_ = (col - HALF) & (W - 1)
        midx[pl.ds(j * L, L)] = (gy_ << 9) + gx_

    ztile = pl.multiple_of(s * PIX_PER_TILE, PIX_PER_TILE)

    for phase in range(2):
        b = 2 * c + phase
        gy_off = pl.multiple_of(b * (2 * HW) + q, CHUNK)
        gx_off = pl.multiple_of(b * (2 * HW) + HW + q, CHUNK)

        for tb in tbs:
            @pl.loop(0, TWORDS // L)
            def _ztbl(j):
                tb[pl.ds(j * L, L)] = zeros16

        @pl.loop(0, PIX_PER_TILE // L)
        def _zero(i):
            w0[pl.ds(i * L, L)] = zeros16

        z0 = pltpu.async_copy(w0, acc.at[pl.ds(ztile, PIX_PER_TILE)], sem_z)
        z0.wait()
        plsc.subcore_barrier()

        def start_inputs(ch):
            d = ch & 1
            a = pltpu.async_copy(
                phi_hbm.at[pl.ds(gy_off + ch * CHUNK, CHUNK)],
                gy_bufs[d], sem_ins[d])
            bcp = pltpu.async_copy(
                phi_hbm.at[pl.ds(gx_off + ch * CHUNK, CHUNK)],
                gx_bufs[d], sem_ins[d])
            return (a, bcp)

        in_pend = {0: start_inputs(0), 1: start_inputs(1)}
        sc_pend = {}

        for ch in range(NCHUNK):
            d = ch & 1
            for cp in in_pend.pop(ch):
                cp.wait()
            if ch - 2 in sc_pend:
                sc_pend.pop(ch - 2).wait()
            gy_buf, gx_buf = gy_bufs[d], gx_bufs[d]
            idx_buf, w_buf = idx_bufs[d], w_bufs[d]

            @pl.loop(0, CHUNK // L, step=NTBL)
            def _compute(i0):
                for j in range(NTBL):
                    i = i0 + j
                    tb = tbs[j]
                    gy = gy_buf[pl.ds(i * L, L)]
                    gx = gx_buf[pl.ds(i * L, L)]
                    ty = gy.astype(jnp.int32)
                    tx = gx.astype(jnp.int32)
                    tyf = ty.astype(jnp.float32)
                    txf = tx.astype(jnp.float32)
                    cy = tyf > gy
                    cx = txf > gx
                    y0i = jnp.where(cy, ty - 1, ty)
                    x0i = jnp.where(cx, tx - 1, tx)
                    wy = gy - jnp.where(cy, tyf - 1.0, tyf)
                    wx = gx - jnp.where(cx, txf - 1.0, txf)
                    uy = 1.0 - wy
                    ux = 1.0 - wx
                    y0 = y0i & (H - 1)
                    x0 = x0i & (W - 1)
                    y1 = (y0i + 1) & (H - 1)
                    x1 = (x0i + 1) & (W - 1)
                    w00 = uy * ux
                    w01 = uy * wx
                    w10 = wy * ux
                    w11 = wy * wx
                    yy0 = (y0 + HALF) & (H - 1)
                    yy1 = (y1 + HALF) & (H - 1)
                    xx0 = (x0 + HALF) & (W - 1)
                    xx1 = (x1 + HALF) & (W - 1)
                    iny0 = yy0 < WIN
                    iny1 = yy1 < WIN
                    inx0 = xx0 < WIN
                    inx1 = xx1 < WIN
                    m00 = iny0 & inx0
                    m01 = iny0 & inx1
                    m10 = iny1 & inx0
                    m11 = iny1 & inx1
                    ry0 = yy0 << 4
                    ry1 = yy1 << 4
                    dump = TBL * L + lane
                    l00 = jnp.where(m00, laneoff + (ry0 + xx0), dump)
                    l01 = jnp.where(m01, laneoff + (ry0 + xx1), dump)
                    l10 = jnp.where(m10, laneoff + (ry1 + xx0), dump)
                    l11 = jnp.where(m11, laneoff + (ry1 + xx1), dump)
                    cur00 = plsc.load_gather(tb, [l00])
                    cur01 = plsc.load_gather(tb, [l01])
                    cur10 = plsc.load_gather(tb, [l10])
                    cur11 = plsc.load_gather(tb, [l11])
                    plsc.store_scatter(tb, [l00],
                                       cur00 + jnp.where(m00, w00, 0.0))
                    plsc.store_scatter(tb, [l01],
                                       cur01 + jnp.where(m01, w01, 0.0))
                    plsc.store_scatter(tb, [l10],
                                       cur10 + jnp.where(m10, w10, 0.0))
                    plsc.store_scatter(tb, [l11],
                                       cur11 + jnp.where(m11, w11, 0.0))
                    r0 = y0 << 9
                    r1 = y1 << 9
                    o = i * L
                    p0 = sink + o + lane
                    fz = jnp.float32(0.0)
                    idx_buf[pl.ds(o, L)] = jnp.where(m00, p0, r0 + x0)
                    idx_buf[pl.ds(CHUNK + o, L)] = jnp.where(
                        m01, p0 + CHUNK, r0 + x1)
                    idx_buf[pl.ds(2 * CHUNK + o, L)] = jnp.where(
                        m10, p0, r1 + x0)
                    idx_buf[pl.ds(3 * CHUNK + o, L)] = jnp.where(
                        m11, p0 + CHUNK, r1 + x1)
                    w_buf[pl.ds(o, L)] = jnp.where(m00, fz, w00)
                    w_buf[pl.ds(CHUNK + o, L)] = jnp.where(m01, fz, w01)
                    w_buf[pl.ds(2 * CHUNK + o, L)] = jnp.where(m10, fz, w10)
                    w_buf[pl.ds(3 * CHUNK + o, L)] = jnp.where(m11, fz, w11)

            sc_pend[ch] = pltpu.async_copy(
                w_buf, acc.at[idx_buf], sem_scs[d], add=True)
            if ch + 2 < NCHUNK:
                in_pend[ch + 2] = start_inputs(ch + 2)

        for ch in sorted(sc_pend):
            sc_pend.pop(ch).wait()

        @pl.loop(0, TBL // L)
        def _merge(j):
            v = tb0[pl.ds(TBL + j * L, L)]
            for k in range(2, L):
                v = v + tb0[pl.ds(k * TBL + j * L, L)]
            for tb in tbs[1:]:
                for k in range(L):
                    v = v + tb[pl.ds(k * TBL + j * L, L)]
            tb0[pl.ds(j * L, L)] = tb0[pl.ds(j * L, L)] + v

        pltpu.sync_copy(tb0.at[pl.ds(0, TBL)], acc.at[midx], add=True)
        plsc.subcore_barrier()

        pltpu.sync_copy(
            acc.at[pl.ds(ztile, PIX_PER_TILE)],
            out_hbm.at[pl.ds(pl.multiple_of(b * HW + q, PIX_PER_TILE),
                             PIX_PER_TILE)],
        )


def _make_splat():
    mesh = plsc.VectorSubcoreMesh(core_axis_name="c", subcore_axis_name="s")
    return pl.kernel(
        _splat_body,
        out_type=jax.ShapeDtypeStruct((P,), jnp.float32),
        mesh=mesh,
        compiler_params=pltpu.CompilerParams(needs_layout_passes=False),
        scratch_types=[
            pltpu.VMEM((CHUNK,), jnp.float32),
            pltpu.VMEM((CHUNK,), jnp.float32),
            pltpu.VMEM((CHUNK,), jnp.float32),
            pltpu.VMEM((CHUNK,), jnp.float32),
            pltpu.VMEM((NPAIR,), jnp.int32),
            pltpu.VMEM((NPAIR,), jnp.int32),
            pltpu.VMEM((NPAIR,), jnp.float32),
            pltpu.VMEM((NPAIR,), jnp.float32),
            pltpu.VMEM((TWORDS,), jnp.float32),
            pltpu.VMEM((TWORDS,), jnp.float32),
            pltpu.VMEM((TWORDS,), jnp.float32),
            pltpu.VMEM((TWORDS,), jnp.float32),
            pltpu.VMEM((TBL,), jnp.int32),
            pltpu.VMEM_SHARED((ACC_WORDS,), jnp.float32),
            pltpu.SemaphoreType.DMA,
            pltpu.SemaphoreType.DMA,
            pltpu.SemaphoreType.DMA,
            pltpu.SemaphoreType.DMA,
            pltpu.SemaphoreType.DMA,
        ],
    )


_splat = _make_splat()


@jax.jit
def kernel(x, phi):
    del x
    cnt = _splat(phi.reshape(-1))
    return cnt.reshape(B, 1, H, W)


# speedup vs baseline: 1.3615x; 1.0670x over previous
import jax
import jax.numpy as jnp
from jax import lax
from jax.experimental import pallas as pl
from jax.experimental.pallas import tpu as pltpu, tpu_sc as plsc

B, H, W = 4, 512, 512
HW = H * W
P = B * HW
NC, NS, L = 2, 16, 16
PIX_PER_TILE = HW // NS
CHUNK = 4096
NCHUNK = PIX_PER_TILE // CHUNK
NPAIR = 4 * CHUNK
WIN = 16
HALF = WIN // 2
TBL = WIN * WIN
NTBL = 4
TWORDS = TBL * L + L
SINK0 = HW
SINKW = 2 * CHUNK
ACC_WORDS = HW + NS * SINKW


def _splat_body(phi_hbm, out_hbm,
                gy0, gy1, gx0, gx1, idx0, idx1, w0, w1,
                tb0, tb1, tb2, tb3, midx,
                acc, sem_in0, sem_in1, sem_sc0, sem_sc1, sem_z):
    c = lax.axis_index("c")
    s = lax.axis_index("s")
    q = s * PIX_PER_TILE
    sink = SINK0 + s * SINKW
    lane = lax.iota(jnp.int32, L)
    zeros16 = jnp.zeros((L,), jnp.float32)
    laneoff = lane << 8

    gy_bufs = (gy0, gy1)
    gx_bufs = (gx0, gx1)
    idx_bufs = (idx0, idx1)
    w_bufs = (w0, w1)
    sem_ins = (sem_in0, sem_in1)
    sem_scs = (sem_sc0, sem_sc1)
    tbs = (tb0, tb1, tb2, tb3)

    @pl.loop(0, TBL // L)
    def _init_tbl(j):
        wcell = j * L + lane
        r = wcell >> 4
        col = wcell & (WIN - 1)
        gy_ = (r - HALF) & (H - 1)
        gx_ = (col - HALF) & (W - 1)
        midx[pl.ds(j * L, L)] = (gy_ << 9) + gx_

    ztile = pl.multiple_of(s * PIX_PER_TILE, PIX_PER_TILE)

    for phase in range(2):
        b = 2 * c + phase
        gy_off = pl.multiple_of(b * (2 * HW) + q, CHUNK)
        gx_off = pl.multiple_of(b * (2 * HW) + HW + q, CHUNK)

        for tb in tbs:
            @pl.loop(0, TWORDS // L)
            def _ztbl(j):
                tb[pl.ds(j * L, L)] = zeros16

        @pl.loop(0, PIX_PER_TILE // L)
        def _zero(i):
            w0[pl.ds(i * L, L)] = zeros16

        z0 = pltpu.async_copy(w0, acc.at[pl.ds(ztile, PIX_PER_TILE)], sem_z)
        z0.wait()
        plsc.subcore_barrier()

        def start_inputs(ch):
            d = ch & 1
            a = pltpu.async_copy(
                phi_hbm.at[pl.ds(gy_off + ch * CHUNK, CHUNK)],
                gy_bufs[d], sem_ins[d])
            bcp = pltpu.async_copy(
                phi_hbm.at[pl.ds(gx_off + ch * CHUNK, CHUNK)],
                gx_bufs[d], sem_ins[d])
            return (a, bcp)

        in_pend = {0: start_inputs(0), 1: start_inputs(1)}
        sc_pend = {}

        for ch in range(NCHUNK):
            d = ch & 1
            for cp in in_pend.pop(ch):
                cp.wait()
            if ch - 2 in sc_pend:
                sc_pend.pop(ch - 2).wait()
            gy_buf, gx_buf = gy_bufs[d], gx_bufs[d]
            idx_buf, w_buf = idx_bufs[d], w_bufs[d]

            @pl.loop(0, CHUNK // L, step=NTBL)
            def _compute(i0):
                for j in range(NTBL):
                    i = i0 + j
                    tb = tbs[j]
                    gy = gy_buf[pl.ds(i * L, L)]
                    gx = gx_buf[pl.ds(i * L, L)]
                    ty = gy.astype(jnp.int32)
                    tx = gx.astype(jnp.int32)
                    tyf = ty.astype(jnp.float32)
                    txf = tx.astype(jnp.float32)
                    cy = tyf > gy
                    cx = txf > gx
                    y0i = jnp.where(cy, ty - 1, ty)
                    x0i = jnp.where(cx, tx - 1, tx)
                    wy = gy - y0i.astype(jnp.float32)
                    wx = gx - x0i.astype(jnp.float32)
                    uy = 1.0 - wy
                    ux = 1.0 - wx
                    y0 = y0i & (H - 1)
                    x0 = x0i & (W - 1)
                    y1 = (y0i + 1) & (H - 1)
                    x1 = (x0i + 1) & (W - 1)
                    w00 = uy * ux
                    w01 = uy * wx
                    w10 = wy * ux
                    w11 = wy * wx
                    yy0 = (y0 + HALF) & (H - 1)
                    yy1 = (y1 + HALF) & (H - 1)
                    xx0 = (x0 + HALF) & (W - 1)
                    xx1 = (x1 + HALF) & (W - 1)
                    m = (yy0 | yy1 | xx0 | xx1) < WIN
                    ry0 = yy0 << 4
                    ry1 = yy1 << 4
                    l00 = laneoff + ((ry0 + xx0) & (TBL - 1))
                    l01 = laneoff + ((ry0 + xx1) & (TBL - 1))
                    l10 = laneoff + ((ry1 + xx0) & (TBL - 1))
                    l11 = laneoff + ((ry1 + xx1) & (TBL - 1))
                    fz = jnp.float32(0.0)
                    cur00 = plsc.load_gather(tb, [l00])
                    cur01 = plsc.load_gather(tb, [l01])
                    cur10 = plsc.load_gather(tb, [l10])
                    cur11 = plsc.load_gather(tb, [l11])
                    plsc.store_scatter(tb, [l00],
                                       cur00 + jnp.where(m, w00, fz))
                    plsc.store_scatter(tb, [l01],
                                       cur01 + jnp.where(m, w01, fz))
                    plsc.store_scatter(tb, [l10],
                                       cur10 + jnp.where(m, w10, fz))
                    plsc.store_scatter(tb, [l11],
                                       cur11 + jnp.where(m, w11, fz))
                    r0 = y0 << 9
                    r1 = y1 << 9
                    o = i * L
                    p0 = sink + o + lane
                    idx_buf[pl.ds(o, L)] = jnp.where(m, p0, r0 + x0)
                    idx_buf[pl.ds(CHUNK + o, L)] = jnp.where(
                        m, p0 + CHUNK, r0 + x1)
                    idx_buf[pl.ds(2 * CHUNK + o, L)] = jnp.where(
                        m, p0, r1 + x0)
                    idx_buf[pl.ds(3 * CHUNK + o, L)] = jnp.where(
                        m, p0 + CHUNK, r1 + x1)
                    w_buf[pl.ds(o, L)] = w00
                    w_buf[pl.ds(CHUNK + o, L)] = w01
                    w_buf[pl.ds(2 * CHUNK + o, L)] = w10
                    w_buf[pl.ds(3 * CHUNK + o, L)] = w11

            sc_pend[ch] = pltpu.async_copy(
                w_buf, acc.at[idx_buf], sem_scs[d], add=True)
            if ch + 2 < NCHUNK:
                in_pend[ch + 2] = start_inputs(ch + 2)

        for ch in sorted(sc_pend):
            sc_pend.pop(ch).wait()

        @pl.loop(0, TBL // L)
        def _merge(j):
            v = tb0[pl.ds(TBL + j * L, L)]
            for k in range(2, L):
                v = v + tb0[pl.ds(k * TBL + j * L, L)]
            for tb in tbs[1:]:
                for k in range(L):
                    v = v + tb[pl.ds(k * TBL + j * L, L)]
            tb0[pl.ds(j * L, L)] = tb0[pl.ds(j * L, L)] + v

        pltpu.sync_copy(tb0.at[pl.ds(0, TBL)], acc.at[midx], add=True)
        plsc.subcore_barrier()

        pltpu.sync_copy(
            acc.at[pl.ds(ztile, PIX_PER_TILE)],
            out_hbm.at[pl.ds(pl.multiple_of(b * HW + q, PIX_PER_TILE),
                             PIX_PER_TILE)],
        )


def _make_splat():
    mesh = plsc.VectorSubcoreMesh(core_axis_name="c", subcore_axis_name="s")
    return pl.kernel(
        _splat_body,
        out_type=jax.ShapeDtypeStruct((P,), jnp.float32),
        mesh=mesh,
        compiler_params=pltpu.CompilerParams(needs_layout_passes=False),
        scratch_types=[
            pltpu.VMEM((CHUNK,), jnp.float32),
            pltpu.VMEM((CHUNK,), jnp.float32),
            pltpu.VMEM((CHUNK,), jnp.float32),
            pltpu.VMEM((CHUNK,), jnp.float32),
            pltpu.VMEM((NPAIR,), jnp.int32),
            pltpu.VMEM((NPAIR,), jnp.int32),
            pltpu.VMEM((NPAIR,), jnp.float32),
            pltpu.VMEM((NPAIR,), jnp.float32),
            pltpu.VMEM((TWORDS,), jnp.float32),
            pltpu.VMEM((TWORDS,), jnp.float32),
            pltpu.VMEM((TWORDS,), jnp.float32),
            pltpu.VMEM((TWORDS,), jnp.float32),
            pltpu.VMEM((TBL,), jnp.int32),
            pltpu.VMEM_SHARED((ACC_WORDS,), jnp.float32),
            pltpu.SemaphoreType.DMA,
            pltpu.SemaphoreType.DMA,
            pltpu.SemaphoreType.DMA,
            pltpu.SemaphoreType.DMA,
            pltpu.SemaphoreType.DMA,
        ],
    )


_splat = _make_splat()


@jax.jit
def kernel(x, phi):
    del x
    cnt = _splat(phi.reshape(-1))
    return cnt.reshape(B, 1, H, W)
